# Initial kernel scaffold; baseline (speedup 1.0000x reference)
import jax
import jax.numpy as jnp
from jax.experimental import pallas as pl


def kernel(x, W, b_lin, bias):
    raise NotImplementedError("write your pallas kernel here")


# speedup vs baseline: 2.5218x; 2.5218x over previous
import functools

import jax
import jax.numpy as jnp
from jax import lax
from jax.experimental import pallas as pl
from jax.experimental.pallas import tpu as pltpu

_E = 16
_G = 4
_GSZ = 4
_KEEP_G = 2
_BLK = 256


def _router_body(x_ref, w_ref, bl_ref, bias_ref, val_ref, idx_ref):
    x = x_ref[...]
    w = w_ref[...]
    logits = lax.dot_general(x, w, (((1,), (1,)), ((), ())),
                             preferred_element_type=jnp.float32)
    logits = logits + bl_ref[...]
    m = jnp.max(logits, axis=1, keepdims=True)
    e = jnp.exp(logits - m)
    scores = e / jnp.sum(e, axis=1, keepdims=True) + bias_ref[...]

    g = [jnp.max(scores[:, gi * _GSZ:(gi + 1) * _GSZ], axis=1, keepdims=True)
         for gi in range(_G)]
    one = jnp.float32(1.0)
    zero = jnp.float32(0.0)
    drop = []
    for gi in range(_G):
        beats = [(g[h] >= g[gi]) if h < gi else (g[h] > g[gi])
                 for h in range(_G) if h != gi]
        rank = sum(jnp.where(t, one, zero) for t in beats)
        drop.append(jnp.where(rank >= 2.0, one, zero))

    eidx = lax.broadcasted_iota(jnp.int32, (_BLK, _E), 1)
    gid = eidx // _GSZ
    dropcol = jnp.where(gid == 0, drop[0],
                        jnp.where(gid == 1, drop[1],
                                  jnp.where(gid == 2, drop[2], drop[3])))
    neg = jnp.float32(-1e30)
    ms = scores + dropcol * neg

    v1 = jnp.max(ms, axis=1, keepdims=True)
    i1 = jnp.min(jnp.where(ms == v1, eidx, _E), axis=1, keepdims=True)
    ms2 = jnp.where(eidx == i1, neg, ms)
    v2 = jnp.max(ms2, axis=1, keepdims=True)
    i2 = jnp.min(jnp.where(ms2 == v2, eidx, _E), axis=1, keepdims=True)

    val_ref[...] = jnp.concatenate([v1, v2], axis=1)
    idx_ref[...] = jnp.concatenate([i1, i2], axis=1)


@jax.jit
def kernel(x, W, b_lin, bias):
    n_tok, dim = x.shape
    grid = (n_tok // _BLK,)
    vals, idxs = pl.pallas_call(
        _router_body,
        grid=grid,
        in_specs=[
            pl.BlockSpec((_BLK, dim), lambda i: (i, 0)),
            pl.BlockSpec((_E, dim), lambda i: (0, 0)),
            pl.BlockSpec((1, _E), lambda i: (0, 0)),
            pl.BlockSpec((1, _E), lambda i: (0, 0)),
        ],
        out_specs=[
            pl.BlockSpec((_BLK, 2), lambda i: (i, 0)),
            pl.BlockSpec((_BLK, 2), lambda i: (i, 0)),
        ],
        out_shape=[
            jax.ShapeDtypeStruct((n_tok, 2), jnp.float32),
            jax.ShapeDtypeStruct((n_tok, 2), jnp.int32),
        ],
    )(x, W, b_lin.reshape(1, _E), bias.reshape(1, _E))
    return vals, idxs


# speedup vs baseline: 3.9723x; 1.5752x over previous
import functools

import jax
import jax.numpy as jnp
from jax import lax
from jax.experimental import pallas as pl
from jax.experimental.pallas import tpu as pltpu

_E = 16
_G = 4
_GSZ = 4
_BLK = 512


def _router_body(x_ref, w_ref, bl_ref, bias_ref, val_ref, idx_ref):
    x = x_ref[...]
    w = w_ref[...]
    logits = lax.dot_general(w, x, (((1,), (1,)), ((), ())),
                             preferred_element_type=jnp.float32)
    logits = logits + bl_ref[:, 0:1]
    m = jnp.max(logits, axis=0, keepdims=True)
    e = jnp.exp(logits - m)
    scores = e / jnp.sum(e, axis=0, keepdims=True) + bias_ref[:, 0:1]

    g = [jnp.max(scores[gi * _GSZ:(gi + 1) * _GSZ, :], axis=0, keepdims=True)
         for gi in range(_G)]
    one = jnp.float32(1.0)
    zero = jnp.float32(0.0)
    drop = []
    for gi in range(_G):
        beats = [(g[h] >= g[gi]) if h < gi else (g[h] > g[gi])
                 for h in range(_G) if h != gi]
        rank = sum(jnp.where(t, one, zero) for t in beats)
        drop.append(jnp.where(rank >= 2.0, one, zero))

    ridx = lax.broadcasted_iota(jnp.int32, (_E, _BLK), 0)
    gidr = ridx // _GSZ
    dropfull = jnp.where(gidr == 0, drop[0],
                         jnp.where(gidr == 1, drop[1],
                                   jnp.where(gidr == 2, drop[2], drop[3])))
    neg = jnp.float32(-1e30)
    ms = scores + dropfull * neg

    v1 = jnp.max(ms, axis=0, keepdims=True)
    i1 = jnp.min(jnp.where(ms == v1, ridx, _E), axis=0, keepdims=True)
    ms2 = jnp.where(ridx == i1, neg, ms)
    v2 = jnp.max(ms2, axis=0, keepdims=True)
    i2 = jnp.min(jnp.where(ms2 == v2, ridx, _E), axis=0, keepdims=True)

    val_ref[0:1, :] = v1
    val_ref[1:2, :] = v2
    idx_ref[0:1, :] = i1
    idx_ref[1:2, :] = i2


@jax.jit
def kernel(x, W, b_lin, bias):
    n_tok, dim = x.shape
    grid = (n_tok // _BLK,)
    vals_t, idx_t = pl.pallas_call(
        _router_body,
        grid=grid,
        in_specs=[
            pl.BlockSpec((_BLK, dim), lambda i: (i, 0)),
            pl.BlockSpec((_E, dim), lambda i: (0, 0)),
            pl.BlockSpec((_E, 128), lambda i: (0, 0)),
            pl.BlockSpec((_E, 128), lambda i: (0, 0)),
        ],
        out_specs=[
            pl.BlockSpec((2, _BLK), lambda i: (0, i)),
            pl.BlockSpec((2, _BLK), lambda i: (0, i)),
        ],
        out_shape=[
            jax.ShapeDtypeStruct((2, n_tok), jnp.float32),
            jax.ShapeDtypeStruct((2, n_tok), jnp.int32),
        ],
    )(x, W,
      jnp.broadcast_to(b_lin[:, None], (_E, 128)),
      jnp.broadcast_to(bias[:, None], (_E, 128)))
    return vals_t.T, idx_t.T
